# Initial kernel scaffold; baseline (speedup 1.0000x reference)
import jax
import jax.numpy as jnp
from jax.experimental import pallas as pl


def kernel(x, edge_index, edge_attr, batch, params):
    raise NotImplementedError("write your pallas kernel here")


# speedup vs baseline: 5.5254x; 5.5254x over previous
import functools

import jax
import jax.numpy as jnp
from jax import lax
from jax.experimental import pallas as pl
from jax.experimental.pallas import tpu as pltpu
from jax.experimental.pallas import tpu_sc as plsc

N = 10000
E = 320000
D = 128
NE = 16
B = 64
STEPS = 5

NS = 16
EPT = E // NS
K = 80
CHUNKS = EPT // K
RPT = 624
ZR = 208
RB = 1000
F32 = jnp.float32


def _leaky(t):
    return jnp.where(t >= 0, t, 0.01 * t)


def _emlp_body(ea_ref, wb_ref, b1_ref, wb2_ref, b2_ref, out_ref):
    t = ea_ref[...] @ wb_ref[...] + b1_ref[...]
    t = _leaky(t)
    t = t @ wb2_ref[...] + b2_ref[...]
    out_ref[...] = jnp.where(t > 0, t, jnp.exp(t) - 1.0)


def _edge_mlp(ea2, wb, b1v, wb2, b2v):
    grid = 20
    rb = (E // 8) // grid
    return pl.pallas_call(
        _emlp_body,
        out_shape=jax.ShapeDtypeStruct((E // 8, 128), F32),
        grid=(grid,),
        in_specs=[
            pl.BlockSpec((rb, 128), lambda i: (i, 0)),
            pl.BlockSpec((128, 256), lambda i: (0, 0)),
            pl.BlockSpec((1, 256), lambda i: (0, 0)),
            pl.BlockSpec((256, 128), lambda i: (0, 0)),
            pl.BlockSpec((1, 128), lambda i: (0, 0)),
        ],
        out_specs=pl.BlockSpec((rb, 128), lambda i: (i, 0)),
    )(ea2, wb, b1v, wb2, b2v)


NEB = 4


def _sc_body(x_hbm, src_hbm, dst_hbm, w_hbm, out_hbm,
             src_c, dst_c, w_c, rows0, rows1, agg,
             gsem0, gsem1, ssem0, ssem1,
             esem0, esem1, esem2, esem3):
    esems = (esem0, esem1, esem2, esem3)
    sid = lax.axis_index("s")
    base = sid * EPT

    zero = jnp.zeros((16,), F32)

    def zrow(r, c):
        for cb in range(D // 16):
            rows0[r, pl.ds(cb * 16, 16)] = zero
        return c

    lax.fori_loop(0, K, zrow, 0)
    for kk in range(7):
        pltpu.sync_copy(rows0, agg.at[pl.ds(sid * RPT + kk * K, K)])
    pltpu.sync_copy(rows0.at[pl.ds(0, 64)],
                    agg.at[pl.ds(sid * RPT + 7 * K, 64)])

    @pl.when(sid == NS - 1)
    def _():
        pltpu.sync_copy(rows0.at[pl.ds(0, 16)], agg.at[pl.ds(NS * RPT, 16)])

    plsc.subcore_barrier()

    rows = (rows0, rows1)
    gsems = (gsem0, gsem1)
    ssems = (ssem0, ssem1)

    def issue_edges(c, eb):
        off = pl.ds(base + c * K, K)
        pltpu.async_copy(src_hbm.at[off], src_c[eb], esems[eb])
        pltpu.async_copy(dst_hbm.at[off], dst_c[eb], esems[eb])
        pltpu.async_copy(w_hbm.at[off], w_c[eb], esems[eb])

    def wait_edges(c, eb):
        off = pl.ds(base + c * K, K)
        pltpu.make_async_copy(src_hbm.at[off], src_c[eb], esems[eb]).wait()
        pltpu.make_async_copy(dst_hbm.at[off], dst_c[eb], esems[eb]).wait()
        pltpu.make_async_copy(w_hbm.at[off], w_c[eb], esems[eb]).wait()

    def issue_gather(eb, b):
        pltpu.async_copy(x_hbm.at[src_c[eb]], rows[b], gsems[b])

    def chunk_work(c, b, eb):
        pltpu.make_async_copy(x_hbm.at[src_c[eb]], rows[b], gsems[b]).wait()

        def grp(g, carry):
            wv = w_c[eb][pl.ds(g * 16, 16)]
            for j in range(16):
                ws = jnp.broadcast_to(wv[j], (16,))
                e = g * 16 + j
                for cb in range(D // 16):
                    sl = pl.ds(cb * 16, 16)
                    rows[b][e, sl] = rows[b][e, sl] * ws
            return carry

        lax.fori_loop(0, K // 16, grp, 0)
        pltpu.async_copy(rows[b], agg.at[dst_c[eb]], ssems[b], add=True)

    for c in range(NEB):
        issue_edges(c, c)
    wait_edges(0, 0)
    issue_gather(0, 0)
    wait_edges(1, 1)
    issue_gather(1, 1)

    def loop_body(k, carry):
        for j in range(NEB):
            c = NEB * k + j
            bb = j % 2
            chunk_work(c, bb, j)

            @pl.when(c + 2 <= CHUNKS - 1)
            def _():
                pltpu.make_async_copy(rows[bb], agg.at[dst_c[j]],
                                      ssems[bb]).wait()

                @pl.when(c + NEB <= CHUNKS - 1)
                def _():
                    issue_edges(c + NEB, j)

                wait_edges(c + 2, (j + 2) % NEB)
                issue_gather((j + 2) % NEB, bb)
        return carry

    lax.fori_loop(0, (CHUNKS - 2) // NEB, loop_body, 0)
    chunk_work(CHUNKS - 2, 0, (CHUNKS - 2) % NEB)
    chunk_work(CHUNKS - 1, 1, (CHUNKS - 1) % NEB)
    pltpu.make_async_copy(rows0, agg.at[dst_c[0]], ssem0).wait()
    pltpu.make_async_copy(rows1, agg.at[dst_c[0]], ssem1).wait()
    plsc.subcore_barrier()

    sl = pl.ds(sid * RPT, RPT)
    pltpu.sync_copy(agg.at[sl], out_hbm.at[sl])

    @pl.when(sid == NS - 1)
    def _():
        tl = pl.ds(NS * RPT, 16)
        pltpu.sync_copy(agg.at[tl], out_hbm.at[tl])


_sc_mesh = plsc.VectorSubcoreMesh(core_axis_name="c", subcore_axis_name="s",
                                  num_cores=1)


def _sc_sparse(x, src, dst, w):
    kern = pl.kernel(
        _sc_body,
        out_type=jax.ShapeDtypeStruct((N, D), F32),
        mesh=_sc_mesh,
        scratch_types=[
            [pltpu.VMEM((K,), jnp.int32) for _ in range(NEB)],
            [pltpu.VMEM((K,), jnp.int32) for _ in range(NEB)],
            [pltpu.VMEM((K,), F32) for _ in range(NEB)],
            pltpu.VMEM((K, D), F32),
            pltpu.VMEM((K, D), F32),
            pltpu.VMEM_SHARED((N, D), F32),
            pltpu.SemaphoreType.DMA,
            pltpu.SemaphoreType.DMA,
            pltpu.SemaphoreType.DMA,
            pltpu.SemaphoreType.DMA,
            pltpu.SemaphoreType.DMA,
            pltpu.SemaphoreType.DMA,
            pltpu.SemaphoreType.DMA,
            pltpu.SemaphoreType.DMA,
        ],
    )
    return kern(x, src, dst, w)


def _dense_body(p_ref, x_ref, w1t_ref, b1_ref, w2t_ref, b2_ref,
                gam_ref, bet_ref, out_ref, t_ref):
    w1t = w1t_ref[...]
    b1 = b1_ref[...]
    w2t = w2t_ref[...]
    b2 = b2_ref[...]

    def blk(i, carry):
        s1, s2 = carry
        sl = pl.ds(i * RB, RB)
        t = p_ref[sl, :] + x_ref[sl, :]
        t = _leaky(t @ w1t + b1)
        t = t @ w2t + b2
        t = _leaky(t)
        t_ref[sl, :] = t
        return (s1 + jnp.sum(t, axis=0, keepdims=True),
                s2 + jnp.sum(t * t, axis=0, keepdims=True))

    s1, s2 = lax.fori_loop(0, N // RB, blk,
                           (jnp.zeros((1, D), F32), jnp.zeros((1, D), F32)))
    mu = s1 / N
    var = s2 / N - mu * mu
    scale = gam_ref[...] * lax.rsqrt(var + 1e-5)
    shift = bet_ref[...] - mu * scale

    def blk2(i, carry):
        sl = pl.ds(i * RB, RB)
        out_ref[sl, :] = t_ref[sl, :] * scale + shift
        return carry

    lax.fori_loop(0, N // RB, blk2, 0)


def _dense(part, x, w1t, b1, w2t, b2, gam, bet):
    return pl.pallas_call(
        _dense_body,
        out_shape=jax.ShapeDtypeStruct((N, D), F32),
        scratch_shapes=[pltpu.VMEM((N, D), F32)],
    )(part, x, w1t, b1, w2t, b2, gam, bet)


def _s2s_body(h_ref, b_ref, wih0_ref, whh0_ref, bs0_ref,
              wih1_ref, whh1_ref, bs1_ref, out_ref, s_ref, ex_ref):
    def mks(i, carry):
        sl = pl.ds(i * RB, RB)
        bc = b_ref[sl, :]
        io = lax.broadcasted_iota(jnp.int32, (RB, D), 1)
        s_ref[sl, :] = (bc == io).astype(F32)
        return carry

    lax.fori_loop(0, N // RB, mks, 0)

    wih0 = wih0_ref[...]
    whh0 = whh0_ref[...]
    bs0 = bs0_ref[...]
    wih1 = wih1_ref[...]
    whh1 = whh1_ref[...]
    bs1 = bs1_ref[...]

    def cell(inp, h, c, wih, whh, bs):
        g = inp @ wih + h @ whh + bs
        i = jax.nn.sigmoid(g[:, :D])
        f = jax.nn.sigmoid(g[:, D:2 * D])
        gg = jnp.tanh(g[:, 2 * D:3 * D])
        o = jax.nn.sigmoid(g[:, 3 * D:])
        c2 = f * c + i * gg
        return o * jnp.tanh(c2), c2

    qs = jnp.zeros((B, 2 * D), F32)
    h0 = jnp.zeros((B, D), F32)
    c0 = jnp.zeros((B, D), F32)
    h1 = jnp.zeros((B, D), F32)
    c1 = jnp.zeros((B, D), F32)

    for _ in range(STEPS):
        h0, c0 = cell(qs, h0, c0, wih0, whh0, bs0)
        h1, c1 = cell(h0, h1, c1, wih1, whh1, bs1)
        q = h1
        qpad = jnp.concatenate([q, jnp.zeros((D - B, D), F32)], axis=0)

        def p1(i, emax):
            sl = pl.ds(i * RB, RB)
            xq = lax.dot_general(h_ref[sl, :], qpad,
                                 (((1,), (1,)), ((), ())))
            ex_ref[sl, :] = xq
            m = jnp.where(s_ref[sl, :] > 0, xq, -jnp.inf)
            return jnp.maximum(emax, jnp.max(m, axis=0, keepdims=True))

        emax = lax.fori_loop(0, N // RB, p1, jnp.full((1, D), -jnp.inf, F32))
        emax = jnp.where(emax == -jnp.inf, 0.0, emax)

        def p2(i, den):
            sl = pl.ds(i * RB, RB)
            ex = jnp.exp(ex_ref[sl, :] - emax) * s_ref[sl, :]
            ex_ref[sl, :] = ex
            return den + jnp.sum(ex, axis=0, keepdims=True)

        den = lax.fori_loop(0, N // RB, p2, jnp.zeros((1, D), F32))
        rden = 1.0 / (den + 1e-16)

        def p3(i, r):
            sl = pl.ds(i * RB, RB)
            a = ex_ref[sl, :] * rden
            return r + lax.dot_general(a, h_ref[sl, :],
                                       (((0,), (0,)), ((), ())))

        r = lax.fori_loop(0, N // RB, p3, jnp.zeros((D, D), F32))
        qs = jnp.concatenate([q, r[:B, :]], axis=1)

    out_ref[...] = qs


def _set2set(h, bcol, wih0t, whh0t, bs0, wih1t, whh1t, bs1):
    return pl.pallas_call(
        _s2s_body,
        out_shape=jax.ShapeDtypeStruct((B, 2 * D), F32),
        scratch_shapes=[pltpu.VMEM((N, D), F32), pltpu.VMEM((N, D), F32)],
    )(h, bcol, wih0t, whh0t, bs0, wih1t, whh1t, bs1)


def kernel(x, edge_index, edge_attr, batch, params):
    src = edge_index[0]
    dst = edge_index[1]

    ea2 = edge_attr.reshape(E // 8, 8 * NE)
    eye8 = jnp.eye(8, dtype=F32)
    wb = jnp.concatenate(
        [jnp.kron(eye8, params[f'We1_{l}'].T) for l in range(4)], axis=1)
    b1v = jnp.concatenate(
        [jnp.tile(params[f'be1_{l}'], 8) for l in range(4)]).reshape(1, 256)
    wb2 = jnp.zeros((256, 128), F32)
    b2v = jnp.zeros((128,), F32)
    for l in range(4):
        wb2 = wb2.at[64 * l:64 * l + 64, 8 * l:8 * l + 8].set(
            jnp.kron(eye8, params[f'We2_{l}'].T))
        b2v = b2v.at[8 * l:8 * l + 8].set(params[f'be2_{l}'][0])
    b2v = b2v.reshape(1, 128)

    wpk = _edge_mlp(ea2, wb, b1v, wb2, b2v)
    w_all = (wpk[:, :32].reshape(E // 8, 4, 8)
             .transpose(1, 0, 2).reshape(4, E))

    h = x
    for l in range(4):
        part = _sc_sparse(h, src, dst, w_all[l])
        h = _dense(part, h,
                   params[f'W1_{l}'].T, params[f'b1_{l}'].reshape(1, D),
                   params[f'W2_{l}'].T, params[f'b2_{l}'].reshape(1, D),
                   params[f'gamma_{l}'].reshape(1, D),
                   params[f'beta_{l}'].reshape(1, D))

    bs0 = (params['bih0'] + params['bhh0']).reshape(1, 4 * D)
    bs1 = (params['bih1'] + params['bhh1']).reshape(1, 4 * D)
    return _set2set(h, batch.reshape(N, 1),
                    params['Wih0'].T, params['Whh0'].T, bs0,
                    params['Wih1'].T, params['Whh1'].T, bs1)


# speedup vs baseline: 6.1469x; 1.1125x over previous
import functools

import jax
import jax.numpy as jnp
from jax import lax
from jax.experimental import pallas as pl
from jax.experimental.pallas import tpu as pltpu
from jax.experimental.pallas import tpu_sc as plsc

N = 10000
E = 320000
D = 128
NE = 16
B = 64
STEPS = 5

NS = 16
EPT = E // NS
K = 80
CHUNKS = EPT // K
RPT = 624
ZR = 208
RB = 1000
F32 = jnp.float32


def _leaky(t):
    return jnp.where(t >= 0, t, 0.01 * t)


def _emlp_body(ea_ref, wb_ref, b1_ref, wb2_ref, b2_ref, out_ref):
    t = ea_ref[...] @ wb_ref[...] + b1_ref[...]
    t = _leaky(t)
    t = t @ wb2_ref[...] + b2_ref[...]
    out_ref[...] = jnp.where(t > 0, t, jnp.exp(t) - 1.0)


def _edge_mlp(ea2, wb, b1v, wb2, b2v):
    grid = 20
    rb = (E // 8) // grid
    return pl.pallas_call(
        _emlp_body,
        out_shape=jax.ShapeDtypeStruct((E // 8, 128), F32),
        grid=(grid,),
        in_specs=[
            pl.BlockSpec((rb, 128), lambda i: (i, 0)),
            pl.BlockSpec((128, 256), lambda i: (0, 0)),
            pl.BlockSpec((1, 256), lambda i: (0, 0)),
            pl.BlockSpec((256, 128), lambda i: (0, 0)),
            pl.BlockSpec((1, 128), lambda i: (0, 0)),
        ],
        out_specs=pl.BlockSpec((rb, 128), lambda i: (i, 0)),
    )(ea2, wb, b1v, wb2, b2v)


NB = 4


def _sc_body(x_hbm, pk_hbm, w_hbm, out_hbm,
             pk_c, w_c, rows, gidx, sidx, agg,
             gsems, ssems, esems):
    sid = lax.axis_index("s")
    base = sid * EPT

    def issue_edges(c, eb):
        off = pl.ds(base + c * K, K)
        pltpu.async_copy(pk_hbm.at[off], pk_c[eb], esems[eb])
        pltpu.async_copy(w_hbm.at[off], w_c[eb], esems[eb])

    def wait_edges(c, eb):
        off = pl.ds(base + c * K, K)
        pltpu.make_async_copy(pk_hbm.at[off], pk_c[eb], esems[eb]).wait()
        pltpu.make_async_copy(w_hbm.at[off], w_c[eb], esems[eb]).wait()

    for c in range(NB):
        issue_edges(c, c)

    zero = jnp.zeros((16,), F32)

    def zrow(r, c):
        for cb in range(D // 16):
            rows[0][r, pl.ds(cb * 16, 16)] = zero
        return c

    lax.fori_loop(0, K, zrow, 0)
    for kk in range(7):
        pltpu.sync_copy(rows[0], agg.at[pl.ds(sid * RPT + kk * K, K)])
    pltpu.sync_copy(rows[0].at[pl.ds(0, 64)],
                    agg.at[pl.ds(sid * RPT + 7 * K, 64)])

    @pl.when(sid == NS - 1)
    def _():
        pltpu.sync_copy(rows[0].at[pl.ds(0, 16)],
                        agg.at[pl.ds(NS * RPT, 16)])

    plsc.subcore_barrier()

    shift16 = jnp.full((16,), 16, jnp.int32)
    mask16 = jnp.full((16,), 0xFFFF, jnp.int32)

    def set_gidx(c, j):
        for g in range(K // 16):
            sl = pl.ds(g * 16, 16)
            gidx[j][sl] = jax.lax.shift_right_logical(pk_c[j][sl], shift16)

    def issue_gather(j):
        pltpu.async_copy(x_hbm.at[gidx[j]], rows[j], gsems[j])

    def chunk_work(c, j):
        pltpu.make_async_copy(x_hbm.at[gidx[j]], rows[j], gsems[j]).wait()

        def grp(g, carry):
            sl = pl.ds(g * 16, 16)
            sidx[j][sl] = jax.lax.bitwise_and(pk_c[j][sl], mask16)
            wv = w_c[j][sl]
            for t in range(16):
                ws = jnp.broadcast_to(wv[t], (16,))
                e = g * 16 + t
                for cb in range(D // 16):
                    cs = pl.ds(cb * 16, 16)
                    rows[j][e, cs] = rows[j][e, cs] * ws
            return carry

        lax.fori_loop(0, K // 16, grp, 0)
        pltpu.async_copy(rows[j], agg.at[sidx[j]], ssems[j], add=True)

    wait_edges(0, 0)
    set_gidx(0, 0)
    issue_gather(0)
    wait_edges(1, 1)
    set_gidx(1, 1)
    issue_gather(1)

    def loop_body(k, carry):
        for j in range(NB):
            c = NB * k + j
            chunk_work(c, j)

            @pl.when(c + 2 <= CHUNKS - 1)
            def _():
                jn = (j + 2) % NB

                @pl.when(c >= 2)
                def _():
                    pltpu.make_async_copy(rows[jn], agg.at[sidx[jn]],
                                          ssems[jn]).wait()

                @pl.when(c + NB <= CHUNKS - 1)
                def _():
                    issue_edges(c + NB, j)

                wait_edges(c + 2, jn)
                set_gidx(c + 2, jn)
                issue_gather(jn)
        return carry

    lax.fori_loop(0, (CHUNKS - 2) // NB, loop_body, 0)
    chunk_work(CHUNKS - 2, (CHUNKS - 2) % NB)
    chunk_work(CHUNKS - 1, (CHUNKS - 1) % NB)
    for j in range(NB):
        pltpu.make_async_copy(rows[j], agg.at[sidx[0]], ssems[j]).wait()
    plsc.subcore_barrier()

    sl = pl.ds(sid * RPT, RPT)
    pltpu.sync_copy(agg.at[sl], out_hbm.at[sl])

    @pl.when(sid == NS - 1)
    def _():
        tl = pl.ds(NS * RPT, 16)
        pltpu.sync_copy(agg.at[tl], out_hbm.at[tl])


_sc_mesh = plsc.VectorSubcoreMesh(core_axis_name="c", subcore_axis_name="s",
                                  num_cores=1)


def _sc_sparse(x, packed, w):
    kern = pl.kernel(
        _sc_body,
        out_type=jax.ShapeDtypeStruct((N, D), F32),
        mesh=_sc_mesh,
        scratch_types=[
            [pltpu.VMEM((K,), jnp.int32) for _ in range(NB)],
            [pltpu.VMEM((K,), F32) for _ in range(NB)],
            [pltpu.VMEM((K, D), F32) for _ in range(NB)],
            [pltpu.VMEM((K,), jnp.int32) for _ in range(NB)],
            [pltpu.VMEM((K,), jnp.int32) for _ in range(NB)],
            pltpu.VMEM_SHARED((N, D), F32),
            [pltpu.SemaphoreType.DMA for _ in range(NB)],
            [pltpu.SemaphoreType.DMA for _ in range(NB)],
            [pltpu.SemaphoreType.DMA for _ in range(NB)],
        ],
    )
    return kern(x, packed, w)


def _dense_body(p_ref, x_ref, w1t_ref, b1_ref, w2t_ref, b2_ref,
                gam_ref, bet_ref, out_ref, t_ref):
    w1t = w1t_ref[...]
    b1 = b1_ref[...]
    w2t = w2t_ref[...]
    b2 = b2_ref[...]

    def blk(i, carry):
        s1, s2 = carry
        sl = pl.ds(i * RB, RB)
        t = p_ref[sl, :] + x_ref[sl, :]
        t = _leaky(t @ w1t + b1)
        t = t @ w2t + b2
        t = _leaky(t)
        t_ref[sl, :] = t
        return (s1 + jnp.sum(t, axis=0, keepdims=True),
                s2 + jnp.sum(t * t, axis=0, keepdims=True))

    s1, s2 = lax.fori_loop(0, N // RB, blk,
                           (jnp.zeros((1, D), F32), jnp.zeros((1, D), F32)))
    mu = s1 / N
    var = s2 / N - mu * mu
    scale = gam_ref[...] * lax.rsqrt(var + 1e-5)
    shift = bet_ref[...] - mu * scale

    def blk2(i, carry):
        sl = pl.ds(i * RB, RB)
        out_ref[sl, :] = t_ref[sl, :] * scale + shift
        return carry

    lax.fori_loop(0, N // RB, blk2, 0)


def _dense(part, x, w1t, b1, w2t, b2, gam, bet):
    return pl.pallas_call(
        _dense_body,
        out_shape=jax.ShapeDtypeStruct((N, D), F32),
        scratch_shapes=[pltpu.VMEM((N, D), F32)],
    )(part, x, w1t, b1, w2t, b2, gam, bet)


def _s2s_body(h_ref, b_ref, wih0_ref, whh0_ref, bs0_ref,
              wih1_ref, whh1_ref, bs1_ref, out_ref, s_ref, ex_ref):
    def mks(i, carry):
        sl = pl.ds(i * RB, RB)
        bc = b_ref[sl, :]
        io = lax.broadcasted_iota(jnp.int32, (RB, D), 1)
        s_ref[sl, :] = (bc == io).astype(F32)
        return carry

    lax.fori_loop(0, N // RB, mks, 0)

    wih0 = wih0_ref[...]
    whh0 = whh0_ref[...]
    bs0 = bs0_ref[...]
    wih1 = wih1_ref[...]
    whh1 = whh1_ref[...]
    bs1 = bs1_ref[...]

    def cell(inp, h, c, wih, whh, bs):
        g = inp @ wih + h @ whh + bs
        i = jax.nn.sigmoid(g[:, :D])
        f = jax.nn.sigmoid(g[:, D:2 * D])
        gg = jnp.tanh(g[:, 2 * D:3 * D])
        o = jax.nn.sigmoid(g[:, 3 * D:])
        c2 = f * c + i * gg
        return o * jnp.tanh(c2), c2

    qs = jnp.zeros((B, 2 * D), F32)
    h0 = jnp.zeros((B, D), F32)
    c0 = jnp.zeros((B, D), F32)
    h1 = jnp.zeros((B, D), F32)
    c1 = jnp.zeros((B, D), F32)

    for _ in range(STEPS):
        h0, c0 = cell(qs, h0, c0, wih0, whh0, bs0)
        h1, c1 = cell(h0, h1, c1, wih1, whh1, bs1)
        q = h1
        qpad = jnp.concatenate([q, jnp.zeros((D - B, D), F32)], axis=0)

        def p1(i, emax):
            sl = pl.ds(i * RB, RB)
            xq = lax.dot_general(h_ref[sl, :], qpad,
                                 (((1,), (1,)), ((), ())))
            ex_ref[sl, :] = xq
            m = jnp.where(s_ref[sl, :] > 0, xq, -jnp.inf)
            return jnp.maximum(emax, jnp.max(m, axis=0, keepdims=True))

        emax = lax.fori_loop(0, N // RB, p1, jnp.full((1, D), -jnp.inf, F32))
        emax = jnp.where(emax == -jnp.inf, 0.0, emax)

        def p2(i, den):
            sl = pl.ds(i * RB, RB)
            ex = jnp.exp(ex_ref[sl, :] - emax) * s_ref[sl, :]
            ex_ref[sl, :] = ex
            return den + jnp.sum(ex, axis=0, keepdims=True)

        den = lax.fori_loop(0, N // RB, p2, jnp.zeros((1, D), F32))
        rden = 1.0 / (den + 1e-16)

        def p3(i, r):
            sl = pl.ds(i * RB, RB)
            a = ex_ref[sl, :] * rden
            return r + lax.dot_general(a, h_ref[sl, :],
                                       (((0,), (0,)), ((), ())))

        r = lax.fori_loop(0, N // RB, p3, jnp.zeros((D, D), F32))
        qs = jnp.concatenate([q, r[:B, :]], axis=1)

    out_ref[...] = qs


def _set2set(h, bcol, wih0t, whh0t, bs0, wih1t, whh1t, bs1):
    return pl.pallas_call(
        _s2s_body,
        out_shape=jax.ShapeDtypeStruct((B, 2 * D), F32),
        scratch_shapes=[pltpu.VMEM((N, D), F32), pltpu.VMEM((N, D), F32)],
    )(h, bcol, wih0t, whh0t, bs0, wih1t, whh1t, bs1)


def kernel(x, edge_index, edge_attr, batch, params):
    src = edge_index[0]
    dst = edge_index[1]
    packed = (src << 16) | dst

    ea2 = edge_attr.reshape(E // 8, 8 * NE)
    eye8 = jnp.eye(8, dtype=F32)
    wb = jnp.concatenate(
        [jnp.kron(eye8, params[f'We1_{l}'].T) for l in range(4)], axis=1)
    b1v = jnp.concatenate(
        [jnp.tile(params[f'be1_{l}'], 8) for l in range(4)]).reshape(1, 256)
    wb2 = jnp.zeros((256, 128), F32)
    b2v = jnp.zeros((128,), F32)
    for l in range(4):
        wb2 = wb2.at[64 * l:64 * l + 64, 8 * l:8 * l + 8].set(
            jnp.kron(eye8, params[f'We2_{l}'].T))
        b2v = b2v.at[8 * l:8 * l + 8].set(params[f'be2_{l}'][0])
    b2v = b2v.reshape(1, 128)

    wpk = _edge_mlp(ea2, wb, b1v, wb2, b2v)
    w_all = (wpk[:, :32].reshape(E // 8, 4, 8)
             .transpose(1, 0, 2).reshape(4, E))

    h = x
    for l in range(4):
        part = _sc_sparse(h, packed, w_all[l])
        h = _dense(part, h,
                   params[f'W1_{l}'].T, params[f'b1_{l}'].reshape(1, D),
                   params[f'W2_{l}'].T, params[f'b2_{l}'].reshape(1, D),
                   params[f'gamma_{l}'].reshape(1, D),
                   params[f'beta_{l}'].reshape(1, D))

    bs0 = (params['bih0'] + params['bhh0']).reshape(1, 4 * D)
    bs1 = (params['bih1'] + params['bhh1']).reshape(1, 4 * D)
    return _set2set(h, batch.reshape(N, 1),
                    params['Wih0'].T, params['Whh0'].T, bs0,
                    params['Wih1'].T, params['Whh1'].T, bs1)


# speedup vs baseline: 8.9592x; 1.4575x over previous
import functools

import jax
import jax.numpy as jnp
from jax import lax
from jax.experimental import pallas as pl
from jax.experimental.pallas import tpu as pltpu
from jax.experimental.pallas import tpu_sc as plsc

N = 10000
E = 320000
D = 128
NE = 16
B = 64
STEPS = 5

NC = 2
NS = 16
NW = NC * NS
EPT = E // NW
K = 80
CHUNKS = EPT // K
RPT = 624
ZR = 208
RB = 1000
F32 = jnp.float32


def _leaky(t):
    return jnp.where(t >= 0, t, 0.01 * t)


def _emlp_body(ea_ref, wb_ref, b1_ref, wb2_ref, b2_ref, out_ref):
    t = ea_ref[...] @ wb_ref[...] + b1_ref[...]
    t = _leaky(t)
    t = t @ wb2_ref[...] + b2_ref[...]
    out_ref[...] = jnp.where(t > 0, t, jnp.exp(t) - 1.0)


def _edge_mlp(ea2, wb, b1v, wb2, b2v):
    grid = 20
    rb = (E // 8) // grid
    return pl.pallas_call(
        _emlp_body,
        out_shape=jax.ShapeDtypeStruct((E // 8, 128), F32),
        grid=(grid,),
        in_specs=[
            pl.BlockSpec((rb, 128), lambda i: (i, 0)),
            pl.BlockSpec((128, 256), lambda i: (0, 0)),
            pl.BlockSpec((1, 256), lambda i: (0, 0)),
            pl.BlockSpec((256, 128), lambda i: (0, 0)),
            pl.BlockSpec((1, 128), lambda i: (0, 0)),
        ],
        out_specs=pl.BlockSpec((rb, 128), lambda i: (i, 0)),
    )(ea2, wb, b1v, wb2, b2v)


NB = 4


def _sc_body(x_hbm, pk_hbm, w_hbm, out_hbm,
             pk_c, w_c, rows, gidx, sidx, agg,
             gsems, ssems, esems):
    cid = lax.axis_index("c")
    sid = lax.axis_index("s")
    base = (sid * NC + cid) * EPT

    def issue_edges(c, eb):
        off = pl.ds(base + c * K, K)
        pltpu.async_copy(pk_hbm.at[off], pk_c[eb], esems[eb])
        pltpu.async_copy(w_hbm.at[off], w_c[eb], esems[eb])

    def wait_edges(c, eb):
        off = pl.ds(base + c * K, K)
        pltpu.make_async_copy(pk_hbm.at[off], pk_c[eb], esems[eb]).wait()
        pltpu.make_async_copy(w_hbm.at[off], w_c[eb], esems[eb]).wait()

    for c in range(NB):
        issue_edges(c, c)

    zero = jnp.zeros((16,), F32)

    def zrow(r, c):
        for cb in range(D // 16):
            rows[0][r, pl.ds(cb * 16, 16)] = zero
        return c

    lax.fori_loop(0, K, zrow, 0)
    for kk in range(7):
        pltpu.sync_copy(rows[0], agg.at[pl.ds(sid * RPT + kk * K, K)])
    pltpu.sync_copy(rows[0].at[pl.ds(0, 64)],
                    agg.at[pl.ds(sid * RPT + 7 * K, 64)])

    @pl.when(sid == NS - 1)
    def _():
        pltpu.sync_copy(rows[0].at[pl.ds(0, 16)],
                        agg.at[pl.ds(NS * RPT, 16)])

    plsc.subcore_barrier()

    shift16 = jnp.full((16,), 16, jnp.int32)
    mask16 = jnp.full((16,), 0xFFFF, jnp.int32)

    def set_gidx(c, j):
        for g in range(K // 16):
            sl = pl.ds(g * 16, 16)
            gidx[j][sl] = jax.lax.shift_right_logical(pk_c[j][sl], shift16)

    def issue_gather(j):
        pltpu.async_copy(x_hbm.at[gidx[j]], rows[j], gsems[j])

    def chunk_work(c, j):
        pltpu.make_async_copy(x_hbm.at[gidx[j]], rows[j], gsems[j]).wait()

        def grp(g, carry):
            sl = pl.ds(g * 16, 16)
            sidx[j][sl] = jax.lax.bitwise_and(pk_c[j][sl], mask16)
            wv = w_c[j][sl]
            for t in range(16):
                ws = jnp.broadcast_to(wv[t], (16,))
                e = g * 16 + t
                for cb in range(D // 16):
                    cs = pl.ds(cb * 16, 16)
                    rows[j][e, cs] = rows[j][e, cs] * ws
            return carry

        lax.fori_loop(0, K // 16, grp, 0)
        pltpu.async_copy(rows[j], agg.at[sidx[j]], ssems[j], add=True)

    wait_edges(0, 0)
    set_gidx(0, 0)
    issue_gather(0)
    wait_edges(1, 1)
    set_gidx(1, 1)
    issue_gather(1)

    NLOOP = (CHUNKS - 5) // NB

    def loop_body(k, carry):
        for j in range(NB):
            c = NB * k + j
            chunk_work(c, j)
            jn = (j + 2) % NB

            @pl.when(c >= 2)
            def _():
                pltpu.make_async_copy(rows[jn], agg.at[sidx[jn]],
                                      ssems[jn]).wait()

            issue_edges(c + NB, j)
            wait_edges(c + 2, jn)
            set_gidx(c + 2, jn)
            issue_gather(jn)
        return carry

    lax.fori_loop(0, NLOOP, loop_body, 0)

    for c in range(NB * NLOOP, CHUNKS):
        j = c % NB
        chunk_work(c, j)
        if c + 2 <= CHUNKS - 1:
            jn = (j + 2) % NB
            pltpu.make_async_copy(rows[jn], agg.at[sidx[jn]],
                                  ssems[jn]).wait()
            if c + NB <= CHUNKS - 1:
                issue_edges(c + NB, j)
            wait_edges(c + 2, jn)
            set_gidx(c + 2, jn)
            issue_gather(jn)

    for j in range(NB):
        pltpu.make_async_copy(rows[j], agg.at[sidx[0]], ssems[j]).wait()
    plsc.subcore_barrier()

    sl = pl.ds(sid * RPT, RPT)
    pltpu.sync_copy(agg.at[sl], out_hbm.at[cid, sl])

    @pl.when(sid == NS - 1)
    def _():
        tl = pl.ds(NS * RPT, 16)
        pltpu.sync_copy(agg.at[tl], out_hbm.at[cid, tl])


_sc_mesh = plsc.VectorSubcoreMesh(core_axis_name="c", subcore_axis_name="s",
                                  num_cores=NC)


def _sc_sparse(x, packed, w):
    kern = pl.kernel(
        _sc_body,
        out_type=jax.ShapeDtypeStruct((NC, N, D), F32),
        mesh=_sc_mesh,
        scratch_types=[
            [pltpu.VMEM((K,), jnp.int32) for _ in range(NB)],
            [pltpu.VMEM((K,), F32) for _ in range(NB)],
            [pltpu.VMEM((K, D), F32) for _ in range(NB)],
            [pltpu.VMEM((K,), jnp.int32) for _ in range(NB)],
            [pltpu.VMEM((K,), jnp.int32) for _ in range(NB)],
            pltpu.VMEM_SHARED((N, D), F32),
            [pltpu.SemaphoreType.DMA for _ in range(NB)],
            [pltpu.SemaphoreType.DMA for _ in range(NB)],
            [pltpu.SemaphoreType.DMA for _ in range(NB)],
        ],
    )
    return kern(x, packed, w)


def _dense_body(p_ref, x_ref, w1t_ref, b1_ref, w2t_ref, b2_ref,
                gam_ref, bet_ref, out_ref, t_ref):
    w1t = w1t_ref[...]
    b1 = b1_ref[...]
    w2t = w2t_ref[...]
    b2 = b2_ref[...]

    def blk(i, carry):
        s1, s2 = carry
        sl = pl.ds(i * RB, RB)
        t = p_ref[0, sl, :] + p_ref[1, sl, :] + x_ref[sl, :]
        t = _leaky(t @ w1t + b1)
        t = t @ w2t + b2
        t = _leaky(t)
        t_ref[sl, :] = t
        return (s1 + jnp.sum(t, axis=0, keepdims=True),
                s2 + jnp.sum(t * t, axis=0, keepdims=True))

    s1, s2 = lax.fori_loop(0, N // RB, blk,
                           (jnp.zeros((1, D), F32), jnp.zeros((1, D), F32)))
    mu = s1 / N
    var = s2 / N - mu * mu
    scale = gam_ref[...] * lax.rsqrt(var + 1e-5)
    shift = bet_ref[...] - mu * scale

    def blk2(i, carry):
        sl = pl.ds(i * RB, RB)
        out_ref[sl, :] = t_ref[sl, :] * scale + shift
        return carry

    lax.fori_loop(0, N // RB, blk2, 0)


def _dense(part, x, w1t, b1, w2t, b2, gam, bet):
    return pl.pallas_call(
        _dense_body,
        out_shape=jax.ShapeDtypeStruct((N, D), F32),
        scratch_shapes=[pltpu.VMEM((N, D), F32)],
    )(part, x, w1t, b1, w2t, b2, gam, bet)


def _s2s_body(h_ref, b_ref, wih0_ref, whh0_ref, bs0_ref,
              wih1_ref, whh1_ref, bs1_ref, out_ref, s_ref, ex_ref):
    def mks(i, carry):
        sl = pl.ds(i * RB, RB)
        bc = b_ref[sl, :]
        io = lax.broadcasted_iota(jnp.int32, (RB, D), 1)
        s_ref[sl, :] = (bc == io).astype(F32)
        return carry

    lax.fori_loop(0, N // RB, mks, 0)

    wih0 = wih0_ref[...]
    whh0 = whh0_ref[...]
    bs0 = bs0_ref[...]
    wih1 = wih1_ref[...]
    whh1 = whh1_ref[...]
    bs1 = bs1_ref[...]

    def cell(inp, h, c, wih, whh, bs):
        g = inp @ wih + h @ whh + bs
        i = jax.nn.sigmoid(g[:, :D])
        f = jax.nn.sigmoid(g[:, D:2 * D])
        gg = jnp.tanh(g[:, 2 * D:3 * D])
        o = jax.nn.sigmoid(g[:, 3 * D:])
        c2 = f * c + i * gg
        return o * jnp.tanh(c2), c2

    qs = jnp.zeros((B, 2 * D), F32)
    h0 = jnp.zeros((B, D), F32)
    c0 = jnp.zeros((B, D), F32)
    h1 = jnp.zeros((B, D), F32)
    c1 = jnp.zeros((B, D), F32)

    for _ in range(STEPS):
        h0, c0 = cell(qs, h0, c0, wih0, whh0, bs0)
        h1, c1 = cell(h0, h1, c1, wih1, whh1, bs1)
        q = h1
        qpad = jnp.concatenate([q, jnp.zeros((D - B, D), F32)], axis=0)

        def p1(i, emax):
            sl = pl.ds(i * RB, RB)
            xq = lax.dot_general(h_ref[sl, :], qpad,
                                 (((1,), (1,)), ((), ())))
            ex_ref[sl, :] = xq
            m = jnp.where(s_ref[sl, :] > 0, xq, -jnp.inf)
            return jnp.maximum(emax, jnp.max(m, axis=0, keepdims=True))

        emax = lax.fori_loop(0, N // RB, p1, jnp.full((1, D), -jnp.inf, F32))
        emax = jnp.where(emax == -jnp.inf, 0.0, emax)

        def p2(i, den):
            sl = pl.ds(i * RB, RB)
            ex = jnp.exp(ex_ref[sl, :] - emax) * s_ref[sl, :]
            ex_ref[sl, :] = ex
            return den + jnp.sum(ex, axis=0, keepdims=True)

        den = lax.fori_loop(0, N // RB, p2, jnp.zeros((1, D), F32))
        rden = 1.0 / (den + 1e-16)

        def p3(i, r):
            sl = pl.ds(i * RB, RB)
            a = ex_ref[sl, :] * rden
            return r + lax.dot_general(a, h_ref[sl, :],
                                       (((0,), (0,)), ((), ())))

        r = lax.fori_loop(0, N // RB, p3, jnp.zeros((D, D), F32))
        qs = jnp.concatenate([q, r[:B, :]], axis=1)

    out_ref[...] = qs


def _set2set(h, bcol, wih0t, whh0t, bs0, wih1t, whh1t, bs1):
    return pl.pallas_call(
        _s2s_body,
        out_shape=jax.ShapeDtypeStruct((B, 2 * D), F32),
        scratch_shapes=[pltpu.VMEM((N, D), F32), pltpu.VMEM((N, D), F32)],
    )(h, bcol, wih0t, whh0t, bs0, wih1t, whh1t, bs1)


def kernel(x, edge_index, edge_attr, batch, params):
    src = edge_index[0]
    dst = edge_index[1]
    packed = (src << 16) | dst

    ea2 = edge_attr.reshape(E // 8, 8 * NE)
    eye8 = jnp.eye(8, dtype=F32)
    wb = jnp.concatenate(
        [jnp.kron(eye8, params[f'We1_{l}'].T) for l in range(4)], axis=1)
    b1v = jnp.concatenate(
        [jnp.tile(params[f'be1_{l}'], 8) for l in range(4)]).reshape(1, 256)
    wb2 = jnp.zeros((256, 128), F32)
    b2v = jnp.zeros((128,), F32)
    for l in range(4):
        wb2 = wb2.at[64 * l:64 * l + 64, 8 * l:8 * l + 8].set(
            jnp.kron(eye8, params[f'We2_{l}'].T))
        b2v = b2v.at[8 * l:8 * l + 8].set(params[f'be2_{l}'][0])
    b2v = b2v.reshape(1, 128)

    wpk = _edge_mlp(ea2, wb, b1v, wb2, b2v)
    w_all = (wpk[:, :32].reshape(E // 8, 4, 8)
             .transpose(1, 0, 2).reshape(4, E))

    h = x
    for l in range(4):
        part = _sc_sparse(h, packed, w_all[l])
        h = _dense(part, h,
                   params[f'W1_{l}'].T, params[f'b1_{l}'].reshape(1, D),
                   params[f'W2_{l}'].T, params[f'b2_{l}'].reshape(1, D),
                   params[f'gamma_{l}'].reshape(1, D),
                   params[f'beta_{l}'].reshape(1, D))

    bs0 = (params['bih0'] + params['bhh0']).reshape(1, 4 * D)
    bs1 = (params['bih1'] + params['bhh1']).reshape(1, 4 * D)
    return _set2set(h, batch.reshape(N, 1),
                    params['Wih0'].T, params['Whh0'].T, bs0,
                    params['Wih1'].T, params['Whh1'].T, bs1)
